# baseline (speedup 1.0000x reference)
import functools

import jax
import jax.numpy as jnp
from jax import lax
from jax.experimental import pallas as pl
from jax.experimental.pallas import tpu as pltpu
from jax.experimental.pallas import tpu_sc as plsc

IDX_MINOR = 128
SUB = 1
NB = 5
LANES = 128
SLIM_ROWS = 2048


@functools.lru_cache(maxsize=None)
def _build(n_idx_rows: int):
    mesh = plsc.VectorSubcoreMesh(core_axis_name="c", subcore_axis_name="s")
    nc, ns = mesh.num_cores, mesh.num_subcores
    nw = nc * ns
    assert n_idx_rows % (nw * SUB * NB) == 0
    idx_rows_per_w = n_idx_rows // nw
    chunks_per_w = idx_rows_per_w // SUB
    groups = chunks_per_w // NB
    rows_per_chunk = SUB * IDX_MINOR
    n_rows = n_idx_rows * IDX_MINOR

    @functools.partial(
        pl.kernel,
        out_type=jax.ShapeDtypeStruct((n_rows, LANES), jnp.float32),
        mesh=mesh,
        compiler_params=pltpu.CompilerParams(use_tc_tiling_on_sc=True),
        scratch_types=[
            pltpu.VMEM((idx_rows_per_w, IDX_MINOR), jnp.int32),
            pltpu.VMEM((NB, rows_per_chunk, LANES), jnp.float32),
            [pltpu.SemaphoreType.DMA] * NB,
            [pltpu.SemaphoreType.DMA] * NB,
        ],
    )
    def gather_kernel(idx_hbm, table_hbm, out_hbm, idx_v, rows_v, gsems, ssems):
        wid = lax.axis_index("s") * nc + lax.axis_index("c")
        row_base = wid * idx_rows_per_w
        pltpu.sync_copy(idx_hbm.at[pl.ds(row_base, idx_rows_per_w)], idx_v)

        def fire_gathers(ci, b):
            for j in range(SUB):
                pltpu.async_copy(
                    table_hbm.at[idx_v.at[ci * SUB + j]],
                    rows_v.at[b].at[pl.ds(j * IDX_MINOR, IDX_MINOR)],
                    gsems[b],
                )

        def drain_gathers(b):
            for j in range(SUB):
                pltpu.make_async_copy(
                    table_hbm.at[idx_v.at[j]],
                    rows_v.at[b].at[pl.ds(j * IDX_MINOR, IDX_MINOR)],
                    gsems[b],
                ).wait()

        def out_slice(ci):
            return out_hbm.at[pl.ds((row_base + ci * SUB) * IDX_MINOR, rows_per_chunk)]

        for b in range(NB):
            fire_gathers(b, b)

        @pl.loop(0, groups - 1)
        def group_loop(t):
            c0 = t * NB
            for b in range(NB):
                drain_gathers(b)
                pltpu.async_copy(rows_v.at[b], out_slice(c0 + b), ssems[b])
            for b in range(NB):
                pltpu.make_async_copy(rows_v.at[b], out_slice(0), ssems[b]).wait()
                fire_gathers(c0 + NB + b, b)

        c0 = (groups - 1) * NB
        for b in range(NB):
            drain_gathers(b)
            pltpu.async_copy(rows_v.at[b], out_slice(c0 + b), ssems[b])
        for b in range(NB):
            pltpu.make_async_copy(rows_v.at[b], out_slice(0), ssems[b]).wait()

    return gather_kernel


def _slim_body(x_ref, o_ref):
    o_ref[...] = x_ref[:, :64]


@functools.lru_cache(maxsize=None)
def _slim(n_rows: int, dim: int):
    assert n_rows % SLIM_ROWS == 0
    return pl.pallas_call(
        _slim_body,
        grid=(n_rows // SLIM_ROWS,),
        in_specs=[pl.BlockSpec((SLIM_ROWS, LANES), lambda i: (i, 0))],
        out_specs=pl.BlockSpec((SLIM_ROWS, dim), lambda i: (i, 0)),
        out_shape=jax.ShapeDtypeStruct((n_rows, dim), jnp.float32),
    )


def kernel(input, weight):
    dim = weight.shape[1]
    idx2d = input.reshape(-1, IDX_MINOR).astype(jnp.int32)
    wpad = jnp.pad(weight, ((0, 0), (0, LANES - dim)))
    wide = _build(idx2d.shape[0])(idx2d, wpad)
    out = _slim(wide.shape[0], dim)(wide)
    return out.reshape(*input.shape, dim)


# speedup vs baseline: 1.7395x; 1.7395x over previous
import functools

import jax
import jax.numpy as jnp
from jax import lax
from jax.experimental import pallas as pl
from jax.experimental.pallas import tpu as pltpu
from jax.experimental.pallas import tpu_sc as plsc

IDX_MINOR = 128
SUB = 1
NB = 5
LANES = 128
SLIM_ROWS = 2048


@functools.lru_cache(maxsize=None)
def _build(n_idx_rows: int):
    mesh = plsc.VectorSubcoreMesh(core_axis_name="c", subcore_axis_name="s")
    nc, ns = mesh.num_cores, mesh.num_subcores
    nw = nc * ns
    assert n_idx_rows % (nw * SUB * NB) == 0
    idx_rows_per_w = n_idx_rows // nw
    chunks_per_w = idx_rows_per_w // SUB
    groups = chunks_per_w // NB
    rows_per_chunk = SUB * IDX_MINOR
    n_rows = n_idx_rows * IDX_MINOR

    @functools.partial(
        pl.kernel,
        out_type=jax.ShapeDtypeStruct((n_rows, LANES), jnp.float32),
        mesh=mesh,
        compiler_params=pltpu.CompilerParams(use_tc_tiling_on_sc=True),
        scratch_types=[
            pltpu.VMEM((idx_rows_per_w, IDX_MINOR), jnp.int32),
            pltpu.VMEM((NB, rows_per_chunk, LANES), jnp.float32),
            [pltpu.SemaphoreType.DMA] * NB,
            [pltpu.SemaphoreType.DMA] * NB,
        ],
    )
    def gather_kernel(idx_hbm, table_hbm, out_hbm, idx_v, rows_v, gsems, ssems):
        wid = lax.axis_index("s") * nc + lax.axis_index("c")
        row_base = wid * idx_rows_per_w
        pltpu.sync_copy(idx_hbm.at[pl.ds(row_base, idx_rows_per_w)], idx_v)

        def fire_gathers(ci, b):
            for j in range(SUB):
                pltpu.async_copy(
                    table_hbm.at[idx_v.at[ci * SUB + j]],
                    rows_v.at[b].at[pl.ds(j * IDX_MINOR, IDX_MINOR)],
                    gsems[b],
                )

        def drain_gathers(b):
            for j in range(SUB):
                pltpu.make_async_copy(
                    table_hbm.at[idx_v.at[j]],
                    rows_v.at[b].at[pl.ds(j * IDX_MINOR, IDX_MINOR)],
                    gsems[b],
                ).wait()

        def out_slice(ci):
            return out_hbm.at[pl.ds((row_base + ci * SUB) * IDX_MINOR, rows_per_chunk)]

        for b in range(NB):
            fire_gathers(b, b)

        @pl.loop(0, groups - 1)
        def group_loop(t):
            c0 = t * NB
            for b in range(NB):
                drain_gathers(b)
                pltpu.async_copy(rows_v.at[b], out_slice(c0 + b), ssems[b])
            for b in range(NB):
                pltpu.make_async_copy(rows_v.at[b], out_slice(0), ssems[b]).wait()
                fire_gathers(c0 + NB + b, b)

        c0 = (groups - 1) * NB
        for b in range(NB):
            drain_gathers(b)
            pltpu.async_copy(rows_v.at[b], out_slice(c0 + b), ssems[b])
        for b in range(NB):
            pltpu.make_async_copy(rows_v.at[b], out_slice(0), ssems[b]).wait()

    return gather_kernel


def _slim_body(x_ref, o_ref):
    o_ref[...] = x_ref[:, :64]


@functools.lru_cache(maxsize=None)
def _slim(n_rows: int, dim: int):
    assert n_rows % SLIM_ROWS == 0
    return pl.pallas_call(
        _slim_body,
        grid=(n_rows // SLIM_ROWS,),
        in_specs=[pl.BlockSpec((SLIM_ROWS, LANES), lambda i: (i, 0))],
        out_specs=pl.BlockSpec((SLIM_ROWS, dim), lambda i: (i, 0)),
        out_shape=jax.ShapeDtypeStruct((n_rows, dim), jnp.float32),
    )


def kernel(input, weight):
    dim = weight.shape[1]
    idx2d = input.reshape(-1, IDX_MINOR).astype(jnp.int32)
    wpad = jnp.pad(weight, ((0, 0), (0, LANES - dim)))
    wide = _build(idx2d.shape[0])(idx2d, wpad)
    return wide[:, :dim].reshape(*input.shape, dim)
